# Initial kernel scaffold; baseline (speedup 1.0000x reference)
import jax
import jax.numpy as jnp
from jax.experimental import pallas as pl


def kernel(x, table):
    raise NotImplementedError("write your pallas kernel here")


# speedup vs baseline: 4.8089x; 4.8089x over previous
import functools

import jax
import jax.numpy as jnp
from jax import lax
from jax.experimental import pallas as pl
from jax.experimental.pallas import tpu as pltpu
from jax.experimental.pallas import tpu_sc as plsc

NUM_EMB = 1000000
D = 32
BATCH = 16384
HIST = 200
B = BATCH * HIST

NC = 2
NS = 16
NW = NC * NS
B_PER_W = B // NW

G = 128
CH = 1024
NG = CH // G
NCHUNK = B_PER_W // CH


@functools.partial(
    pl.kernel,
    mesh=plsc.VectorSubcoreMesh(core_axis_name="c", subcore_axis_name="s"),
    compiler_params=pltpu.CompilerParams(use_tc_tiling_on_sc=False),
    out_type=jax.ShapeDtypeStruct((B, D), jnp.float32),
    scratch_types=[
        pltpu.VMEM((NG, G), jnp.int32),
        pltpu.VMEM((CH, D), jnp.float32),
        pltpu.SemaphoreType.DMA,
    ],
)
def _emb_gather(idx_hbm, table_hbm, out_hbm, idx_v, rows_v, sem):
    wid = lax.axis_index("s") * NC + lax.axis_index("c")
    idx_row0 = wid * (B_PER_W // G)
    out_row0 = wid * B_PER_W

    def body(j, _):
        pltpu.sync_copy(idx_hbm.at[pl.ds(idx_row0 + j * NG, NG)], idx_v)
        copies = [
            pltpu.async_copy(
                table_hbm.at[idx_v.at[g]],
                rows_v.at[pl.ds(g * G, G)],
                sem,
            )
            for g in range(NG)
        ]
        for c in copies:
            c.wait()
        pltpu.sync_copy(rows_v, out_hbm.at[pl.ds(out_row0 + j * CH, CH)])
        return _

    lax.fori_loop(0, NCHUNK, body, None)


def kernel(x, table):
    idx2d = x.astype(jnp.int32).reshape(B // G, G)
    out = _emb_gather(idx2d, table)
    return out.reshape(BATCH, HIST, D)


# speedup vs baseline: 5.0470x; 1.0495x over previous
import functools

import jax
import jax.numpy as jnp
from jax import lax
from jax.experimental import pallas as pl
from jax.experimental.pallas import tpu as pltpu
from jax.experimental.pallas import tpu_sc as plsc

NUM_EMB = 1000000
D = 32
BATCH = 16384
HIST = 200
B = BATCH * HIST

NC = 2
NS = 16
NW = NC * NS
B_PER_W = B // NW

G = 128
CH = 1024
NG = CH // G
NCHUNK = B_PER_W // CH


@functools.partial(
    pl.kernel,
    mesh=plsc.VectorSubcoreMesh(core_axis_name="c", subcore_axis_name="s"),
    compiler_params=pltpu.CompilerParams(use_tc_tiling_on_sc=False),
    out_type=jax.ShapeDtypeStruct((B, D), jnp.float32),
    scratch_types=[
        pltpu.VMEM((2, NG, G), jnp.int32),
        pltpu.VMEM((2, CH, D), jnp.float32),
        pltpu.SemaphoreType.DMA,
        pltpu.SemaphoreType.DMA,
        pltpu.SemaphoreType.DMA,
        pltpu.SemaphoreType.DMA,
        pltpu.SemaphoreType.DMA,
        pltpu.SemaphoreType.DMA,
    ],
)
def _emb_gather(idx_hbm, table_hbm, out_hbm, idx_v, rows_v,
                isem0, isem1, gsem0, gsem1, ssem0, ssem1):
    wid = lax.axis_index("s") * NC + lax.axis_index("c")
    idx_row0 = wid * (B_PER_W // G)
    out_row0 = wid * B_PER_W
    isem = (isem0, isem1)
    gsem = (gsem0, gsem1)
    ssem = (ssem0, ssem1)

    def idx_copy(t, slot):
        return pltpu.make_async_copy(
            idx_hbm.at[pl.ds(idx_row0 + t * NG, NG)], idx_v.at[slot],
            isem[slot])

    def gather_copies(slot):
        return [
            pltpu.make_async_copy(
                table_hbm.at[idx_v.at[slot].at[g]],
                rows_v.at[slot].at[pl.ds(g * G, G)],
                gsem[slot])
            for g in range(NG)
        ]

    def store_copy(t, slot):
        return pltpu.make_async_copy(
            rows_v.at[slot], out_hbm.at[pl.ds(out_row0 + t * CH, CH)],
            ssem[slot])

    idx_copy(0, 0).start()
    idx_copy(0, 0).wait()
    for c in gather_copies(0):
        c.start()
    idx_copy(1, 1).start()

    def pair_body(p, _):
        for b in (0, 1):
            t = 2 * p + b
            o = 1 - b

            @pl.when(t + 1 < NCHUNK)
            def _():
                idx_copy(t + 1, o).wait()

                @pl.when(t >= 1)
                def _():
                    store_copy(t - 1, o).wait()

                for c in gather_copies(o):
                    c.start()

            for c in gather_copies(b):
                c.wait()

            @pl.when(t + 2 < NCHUNK)
            def _():
                idx_copy(t + 2, b).start()

            store_copy(t, b).start()
        return _

    lax.fori_loop(0, NCHUNK // 2, pair_body, None)
    store_copy(0, 0).wait()
    store_copy(0, 1).wait()


def kernel(x, table):
    idx2d = x.astype(jnp.int32).reshape(B // G, G)
    out = _emb_gather(idx2d, table)
    return out.reshape(BATCH, HIST, D)
